# baseline (speedup 1.0000x reference)
import functools
import numpy as np
import jax
import jax.numpy as jnp
from jax import lax
from jax.experimental import pallas as pl
from jax.experimental.pallas import tpu as pltpu

_BN_DIV = np.sqrt(1.0 + 1e-5)
_INTERPRET = False


def _sqdist(src, dst):
    return (jnp.sum(src ** 2, -1)[:, :, None]
            + jnp.sum(dst ** 2, -1)[:, None, :]
            - 2.0 * jnp.einsum('bnc,bmc->bnm', src, dst))


def _gather_rows(points, idx):
    return jax.vmap(lambda p, i: p[i])(points, idx)


def _bn_relu(x, g, be):
    return jax.nn.relu(x / _BN_DIV * g + be)


def _mlp(layers, x):
    for (W, b, g, be) in layers:
        x = _bn_relu(x @ W + b, g, be)
    return x


def _fps_body(npoint, xs_ref, ys_ref, zs_ref, lane_ref, kiota_ref,
              ci_ref, cx_ref, cy_ref, cz_ref):
    xs, ys, zs = xs_ref[...], ys_ref[...], zs_ref[...]
    B, N = xs.shape
    lane = lane_ref[...]
    kiota = kiota_ref[...]

    def body(i, c):
        dist, far, ci, cxa, cya, cza = c
        sel = lane == far
        cx = jnp.sum(jnp.where(sel, xs, 0.0), -1, keepdims=True)
        cy = jnp.sum(jnp.where(sel, ys, 0.0), -1, keepdims=True)
        cz = jnp.sum(jnp.where(sel, zs, 0.0), -1, keepdims=True)
        dx, dy, dz = xs - cx, ys - cy, zs - cz
        d = dx * dx + dy * dy + dz * dz
        dist = jnp.minimum(dist, d)
        m = jnp.max(dist, -1, keepdims=True)
        nfar = jnp.min(jnp.where(dist == m, lane, N), -1, keepdims=True)
        ui = (kiota == i).astype(jnp.int32)
        uf = ui.astype(jnp.float32)
        ci = ci + ui * (far - ci)
        cxa = cxa + uf * (cx - cxa)
        cya = cya + uf * (cy - cya)
        cza = cza + uf * (cz - cza)
        return dist, nfar, ci, cxa, cya, cza

    init = (jnp.full((B, N), 1e10, jnp.float32),
            jnp.zeros((B, 1), jnp.int32),
            jnp.zeros((B, npoint), jnp.int32),
            jnp.zeros((B, npoint), jnp.float32),
            jnp.zeros((B, npoint), jnp.float32),
            jnp.zeros((B, npoint), jnp.float32))
    _, _, ci, cxa, cya, cza = lax.fori_loop(0, npoint, body, init)
    ci_ref[...] = ci
    cx_ref[...] = cxa
    cy_ref[...] = cya
    cz_ref[...] = cza


def _fps(xyz, npoint):
    B, N, _ = xyz.shape
    xs, ys, zs = xyz[..., 0], xyz[..., 1], xyz[..., 2]
    lane = jnp.broadcast_to(jnp.arange(N, dtype=jnp.int32), (B, N))
    kiota = jnp.broadcast_to(jnp.arange(npoint, dtype=jnp.int32), (B, npoint))
    ci, cx, cy, cz = pl.pallas_call(
        functools.partial(_fps_body, npoint),
        grid=(1,),
        in_specs=[pl.BlockSpec((B, N), lambda i: (0, 0))] * 4
        + [pl.BlockSpec((B, npoint), lambda i: (0, 0))],
        out_specs=[pl.BlockSpec((B, npoint), lambda i: (0, 0))] * 4,
        out_shape=[jax.ShapeDtypeStruct((B, npoint), jnp.int32)]
        + [jax.ShapeDtypeStruct((B, npoint), jnp.float32)] * 3,
        interpret=_INTERPRET,
    )(xs, ys, zs, lane, kiota)
    return ci, jnp.stack([cx, cy, cz], -1)


def _ball_query(radius, K, xyz, new_xyz):
    B, S, _ = new_xyz.shape
    N = xyz.shape[1]
    sq = _sqdist(new_xyz, xyz)
    cand = jnp.where(sq > radius ** 2, N,
                     jnp.broadcast_to(jnp.arange(N, dtype=jnp.int32), sq.shape))
    gidx = -lax.top_k(-cand, K)[0]
    first = jnp.broadcast_to(gidx[:, :, :1], gidx.shape)
    return jnp.where(gidx == N, first, gidx)


def _sa_branch(xyz, pts, new_xyz, gidx, layers):
    B, S, K = gidx.shape
    W1, b1, g1, be1 = layers[0]
    C = pts.shape[-1]
    A1 = pts @ W1[:C] + xyz @ W1[C:] + b1
    cW = new_xyz @ W1[C:]
    g = _gather_rows(A1, gidx.reshape(B, S * K)).reshape(B, S, K, -1)
    h = _bn_relu(g - cW[:, :, None, :], g1, be1)
    for (W, b, gg, bb) in layers[1:]:
        h = _bn_relu(h @ W + b, gg, bb)
    return jnp.max(h, axis=2)


def _sa_level(xyz, pts, npoint, radii, Ks, branches):
    fi, new_xyz = _fps(xyz, npoint)
    outs = [_sa_branch(xyz, pts, new_xyz, _ball_query(r, K, xyz, new_xyz), ls)
            for r, K, ls in zip(radii, Ks, branches)]
    return new_xyz, jnp.concatenate(outs, -1)


def _fp(x1, x2, p1, p2, layers):
    B, N, _ = x1.shape
    S = x2.shape[1]
    if S == 1:
        interp = jnp.broadcast_to(p2, (B, N, p2.shape[-1]))
    else:
        d = _sqdist(x1, x2)
        iota = jnp.arange(S, dtype=jnp.int32)[None, None, :]
        dd = d
        idxs, dvs = [], []
        for _ in range(3):
            i = jnp.argmin(dd, -1).astype(jnp.int32)
            dvs.append(jnp.min(dd, -1))
            idxs.append(i)
            dd = jnp.where(iota == i[..., None], jnp.inf, dd)
        d3 = jnp.stack(dvs, -1)
        w = 1.0 / (d3 + 1e-8)
        w = w / jnp.sum(w, -1, keepdims=True)
        rows = [_gather_rows(p2, ii) for ii in idxs]
        interp = (rows[0] * w[..., 0:1] + rows[1] * w[..., 1:2]
                  + rows[2] * w[..., 2:3])
    return _mlp(layers, jnp.concatenate([p1, interp], -1))


def _head_body(h_ref, W1_ref, b1_ref, g1_ref, be1_ref, W2_ref, b2_ref, out_ref):
    x = h_ref[...] @ W1_ref[...] + b1_ref[...]
    x = jax.nn.relu(x / _BN_DIV * g1_ref[...] + be1_ref[...])
    x = x @ W2_ref[...] + b2_ref[...]
    m = jnp.max(x, -1, keepdims=True)
    s = x - m
    out_ref[...] = s - jnp.log(jnp.sum(jnp.exp(s), -1, keepdims=True))


def _head(h, head1, head2):
    B, N, C = h.shape
    W1, b1, g1, be1 = head1
    W2, b2 = head2
    rows = B * N
    BLK = 2048
    hf = h.reshape(rows, C)
    out = pl.pallas_call(
        _head_body,
        grid=(rows // BLK,),
        in_specs=[
            pl.BlockSpec((BLK, C), lambda i: (i, 0)),
            pl.BlockSpec((C, W1.shape[1]), lambda i: (0, 0)),
            pl.BlockSpec((1, W1.shape[1]), lambda i: (0, 0)),
            pl.BlockSpec((1, W1.shape[1]), lambda i: (0, 0)),
            pl.BlockSpec((1, W1.shape[1]), lambda i: (0, 0)),
            pl.BlockSpec((C, W2.shape[1]), lambda i: (0, 0)),
            pl.BlockSpec((1, W2.shape[1]), lambda i: (0, 0)),
        ],
        out_specs=pl.BlockSpec((BLK, W2.shape[1]), lambda i: (i, 0)),
        out_shape=jax.ShapeDtypeStruct((rows, W2.shape[1]), jnp.float32),
        interpret=_INTERPRET,
    )(hf, W1, b1.reshape(1, -1), g1.reshape(1, -1), be1.reshape(1, -1),
      W2, b2.reshape(1, -1))
    return out.reshape(B, N, -1)


@jax.jit
def _forward(xyz, cls_label, params):
    B, C, N = xyz.shape
    x0 = jnp.transpose(xyz, (0, 2, 1))

    x1, f1 = _sa_level(x0, x0, 1024, [0.1, 0.2, 0.4], [32, 64, 128],
                       params['sa1'])
    x2, f2 = _sa_level(x1, f1, 512, [0.4, 0.8], [64, 128], params['sa2'])

    h = jnp.concatenate([x2, f2], -1)
    f3 = jnp.max(_mlp(params['sa3'], h), axis=1, keepdims=True)
    x3 = jnp.zeros((B, 1, 3), jnp.float32)

    f2 = _fp(x2, x3, f2, f3, params['fp3'])
    f1 = _fp(x1, x2, f1, f2, params['fp2'])

    cls_oh = jnp.broadcast_to(cls_label.reshape(B, 1, 1), (B, N, 1))
    p1 = jnp.concatenate([cls_oh, x0, x0], -1)
    f0 = _fp(x0, x1, p1, f1, params['fp1'])

    out = _head(f0, params['head1'], params['head2'])
    l3_points = jnp.transpose(f3, (0, 2, 1))
    return out, l3_points


def kernel(xyz, cls_label, params):
    return _forward(xyz, cls_label, params)


# speedup vs baseline: 6.0894x; 6.0894x over previous
import functools
import numpy as np
import jax
import jax.numpy as jnp
from jax import lax
from jax.experimental import pallas as pl
from jax.experimental.pallas import tpu as pltpu
from jax.experimental.pallas import tpu_sc as plsc

_BN_DIV = np.sqrt(1.0 + 1e-5)
_INTERPRET = False


def _sqdist(src, dst):
    return (jnp.sum(src ** 2, -1)[:, :, None]
            + jnp.sum(dst ** 2, -1)[:, None, :]
            - 2.0 * jnp.einsum('bnc,bmc->bnm', src, dst))


def _gather_rows(points, idx):
    return jax.vmap(lambda p, i: p[i])(points, idx)


def _bn_relu(x, g, be):
    return jax.nn.relu(x / _BN_DIV * g + be)


def _mlp(layers, x):
    for (W, b, g, be) in layers:
        x = _bn_relu(x @ W + b, g, be)
    return x


def _fps_body(npoint, xs_ref, ys_ref, zs_ref, lane_ref, kiota_ref,
              ci_ref, cx_ref, cy_ref, cz_ref):
    xs, ys, zs = xs_ref[...], ys_ref[...], zs_ref[...]
    B, N = xs.shape
    lane = lane_ref[...]
    kiota = kiota_ref[...]

    def body(i, c):
        dist, far, ci, cxa, cya, cza = c
        sel = lane == far
        cx = jnp.sum(jnp.where(sel, xs, 0.0), -1, keepdims=True)
        cy = jnp.sum(jnp.where(sel, ys, 0.0), -1, keepdims=True)
        cz = jnp.sum(jnp.where(sel, zs, 0.0), -1, keepdims=True)
        dx, dy, dz = xs - cx, ys - cy, zs - cz
        d = dx * dx + dy * dy + dz * dz
        dist = jnp.minimum(dist, d)
        m = jnp.max(dist, -1, keepdims=True)
        nfar = jnp.min(jnp.where(dist == m, lane, N), -1, keepdims=True)
        ui = (kiota == i).astype(jnp.int32)
        uf = ui.astype(jnp.float32)
        ci = ci + ui * (far - ci)
        cxa = cxa + uf * (cx - cxa)
        cya = cya + uf * (cy - cya)
        cza = cza + uf * (cz - cza)
        return dist, nfar, ci, cxa, cya, cza

    init = (jnp.full((B, N), 1e10, jnp.float32),
            jnp.zeros((B, 1), jnp.int32),
            jnp.zeros((B, npoint), jnp.int32),
            jnp.zeros((B, npoint), jnp.float32),
            jnp.zeros((B, npoint), jnp.float32),
            jnp.zeros((B, npoint), jnp.float32))
    _, _, ci, cxa, cya, cza = lax.fori_loop(0, npoint, body, init)
    ci_ref[...] = ci
    cx_ref[...] = cxa
    cy_ref[...] = cya
    cz_ref[...] = cza


def _fps(xyz, npoint):
    B, N, _ = xyz.shape
    xs, ys, zs = xyz[..., 0], xyz[..., 1], xyz[..., 2]
    lane = jnp.broadcast_to(jnp.arange(N, dtype=jnp.int32), (B, N))
    kiota = jnp.broadcast_to(jnp.arange(npoint, dtype=jnp.int32), (B, npoint))
    ci, cx, cy, cz = pl.pallas_call(
        functools.partial(_fps_body, npoint),
        grid=(1,),
        in_specs=[pl.BlockSpec((B, N), lambda i: (0, 0))] * 4
        + [pl.BlockSpec((B, npoint), lambda i: (0, 0))],
        out_specs=[pl.BlockSpec((B, npoint), lambda i: (0, 0))] * 4,
        out_shape=[jax.ShapeDtypeStruct((B, npoint), jnp.int32)]
        + [jax.ShapeDtypeStruct((B, npoint), jnp.float32)] * 3,
        interpret=_INTERPRET,
    )(xs, ys, zs, lane, kiota)
    return ci, jnp.stack([cx, cy, cz], -1)


_SC_NB = 16
_SC_NW = 32


def _sc_select(dist2, S, N, branches):
    R = dist2.shape[0]
    Rper = R // _SC_NW
    nblk = Rper // _SC_NB
    nbr = len(branches)
    out_type = [jax.ShapeDtypeStruct((R * K,), jnp.int32) for _, K in branches]
    scratch = ([pltpu.VMEM((_SC_NB, N), jnp.float32)]
               + [pltpu.VMEM((_SC_NB * K + 16,), jnp.int32)
                  for _, K in branches])
    mesh = plsc.VectorSubcoreMesh(core_axis_name="c", subcore_axis_name="s")

    @functools.partial(pl.kernel, out_type=out_type, mesh=mesh,
                       scratch_types=scratch)
    def sel_kernel(dist_hbm, *refs):
        gidx_hbm = refs[0:nbr]
        dist_v = refs[nbr]
        out_v = refs[nbr + 1:nbr + 1 + nbr]
        wid = lax.axis_index("s") * 2 + lax.axis_index("c")
        iota16 = lax.broadcasted_iota(jnp.int32, (16,), 0)

        def do_block(blk, carry0):
            r0 = wid * Rper + blk * _SC_NB
            pltpu.sync_copy(dist_hbm.at[pl.ds(r0, _SC_NB)], dist_v)

            def do_row(rr, carry1):
                base = ((r0 + rr) // S) * N
                basev = iota16 * 0 + base
                for j, (_, K) in enumerate(branches):
                    for w in range(K // 16):
                        out_v[j][pl.ds(rr * K + w * 16, 16)] = basev

                def chunk(c, offs):
                    d = dist_v[rr, pl.ds(c * 16, 16)]
                    nf = iota16 + (base + c * 16)
                    new = []
                    for j, (r2, K) in enumerate(branches):
                        mj = d <= r2
                        mi = mj.astype(jnp.int32)
                        off = offs[j]
                        excl = plsc.cumsum(mi) - mi
                        tgt = jnp.minimum(rr * K + off + excl,
                                          _SC_NB * K + 15)
                        plsc.store_scatter(out_v[j], [tgt], nf, mask=mj)
                        pc = jnp.sum(mi)
                        new.append(jnp.minimum(off + pc, K))
                    return tuple(new)

                lax.fori_loop(0, N // 16, chunk,
                              tuple(jnp.int32(0) for _ in range(nbr)))
                return carry1

            lax.fori_loop(0, _SC_NB, do_row, 0)
            for j, (_, K) in enumerate(branches):
                pltpu.sync_copy(out_v[j].at[pl.ds(0, _SC_NB * K)],
                                gidx_hbm[j].at[pl.ds(r0 * K, _SC_NB * K)])
            return carry0

        lax.fori_loop(0, nblk, do_block, 0)

    return sel_kernel(dist2)


def _sc_gather(table, idx):
    T, D = table.shape
    G = idx.shape[0]
    Gper = G // _SC_NW
    RI = min(2048, 65536 // D)
    RI = min(RI, Gper)
    niter = Gper // RI
    nsub = RI // 128
    idx2 = idx.reshape(G // RI, nsub, 128)

    mesh = plsc.VectorSubcoreMesh(core_axis_name="c", subcore_axis_name="s")

    @functools.partial(
        pl.kernel, out_type=jax.ShapeDtypeStruct((G, D), jnp.float32),
        mesh=mesh,
        scratch_types=[pltpu.VMEM((nsub, 128), jnp.int32),
                       pltpu.VMEM((RI, D), jnp.float32),
                       pltpu.SemaphoreType.DMA],
        compiler_params=pltpu.CompilerParams(use_tc_tiling_on_sc=False))
    def gather_kernel(table_hbm, idx_hbm, out_hbm, idx_v, rows_v, sem):
        wid = lax.axis_index("s") * 2 + lax.axis_index("c")

        def do_iter(it, carry):
            g0 = wid * Gper + it * RI
            pltpu.sync_copy(idx_hbm.at[wid * niter + it], idx_v)
            copies = []
            for j in range(nsub):
                copies.append(pltpu.async_copy(
                    table_hbm.at[idx_v.at[j]],
                    rows_v.at[pl.ds(j * 128, 128)], sem))
            for c in copies:
                c.wait()
            pltpu.sync_copy(rows_v, out_hbm.at[pl.ds(g0, RI)])
            return carry

        lax.fori_loop(0, niter, do_iter, 0)

    return gather_kernel(table, idx2)


def _ball_query_all(radii, Ks, xyz, new_xyz):
    B, S, _ = new_xyz.shape
    N = xyz.shape[1]
    sq = _sqdist(new_xyz, xyz)
    counts = [jnp.minimum(jnp.sum((sq <= r ** 2).astype(jnp.int32), -1), K)
              for r, K in zip(radii, Ks)]
    flats = []
    base = (jnp.arange(B, dtype=jnp.int32) * N)[:, None, None]
    for r, K in zip(radii, Ks):
        cand = jnp.where(sq > r ** 2, N,
                         jnp.broadcast_to(jnp.arange(N, dtype=jnp.int32),
                                          sq.shape))
        g = -lax.top_k(-cand, K)[0]
        g = jnp.where(g == N, 0, g)
        flats.append(g + base)
    return flats, counts


def _sa_branch(xyz, pts, new_xyz, gflat, count, layers):
    B, S, K = gflat.shape
    N = xyz.shape[1]
    W1, b1, g1, be1 = layers[0]
    C = pts.shape[-1]
    A1 = pts @ W1[:C] + xyz @ W1[C:] + b1
    cW = new_xyz @ W1[C:]
    H = A1.shape[-1]
    g = _sc_gather(A1.reshape(B * N, H), gflat.reshape(-1)).reshape(B, S, K, H)
    h = _bn_relu(g - cW[:, :, None, :], g1, be1)
    for (W, b, gg, bb) in layers[1:]:
        h = _bn_relu(h @ W + b, gg, bb)
    valid = (jnp.arange(K, dtype=jnp.int32)[None, None, :]
             < count[:, :, None])
    h = jnp.where(valid[..., None], h, -jnp.inf)
    return jnp.max(h, axis=2)


def _sa_level(xyz, pts, npoint, radii, Ks, branches):
    fi, new_xyz = _fps(xyz, npoint)
    gflats, counts = _ball_query_all(radii, Ks, xyz, new_xyz)
    outs = [_sa_branch(xyz, pts, new_xyz, gf, cnt, ls)
            for gf, cnt, ls in zip(gflats, counts, branches)]
    return new_xyz, jnp.concatenate(outs, -1)


def _fp(x1, x2, p1, p2, layers):
    B, N, _ = x1.shape
    S = x2.shape[1]
    if S == 1:
        interp = jnp.broadcast_to(p2, (B, N, p2.shape[-1]))
    else:
        d = _sqdist(x1, x2)
        iota = jnp.arange(S, dtype=jnp.int32)[None, None, :]
        dd = d
        idxs, dvs = [], []
        for _ in range(3):
            i = jnp.argmin(dd, -1).astype(jnp.int32)
            dvs.append(jnp.min(dd, -1))
            idxs.append(i)
            dd = jnp.where(iota == i[..., None], jnp.inf, dd)
        d3 = jnp.stack(dvs, -1)
        w = 1.0 / (d3 + 1e-8)
        w = w / jnp.sum(w, -1, keepdims=True)
        wmat = (w[..., 0:1] * (iota == idxs[0][..., None])
                + w[..., 1:2] * (iota == idxs[1][..., None])
                + w[..., 2:3] * (iota == idxs[2][..., None]))
        interp = jnp.einsum('bns,bsc->bnc', wmat, p2)
    return _mlp(layers, jnp.concatenate([p1, interp], -1))


def _head_body(h_ref, W1_ref, b1_ref, g1_ref, be1_ref, W2_ref, b2_ref, out_ref):
    x = h_ref[...] @ W1_ref[...] + b1_ref[...]
    x = jax.nn.relu(x / _BN_DIV * g1_ref[...] + be1_ref[...])
    x = x @ W2_ref[...] + b2_ref[...]
    m = jnp.max(x, -1, keepdims=True)
    s = x - m
    out_ref[...] = s - jnp.log(jnp.sum(jnp.exp(s), -1, keepdims=True))


def _head(h, head1, head2):
    B, N, C = h.shape
    W1, b1, g1, be1 = head1
    W2, b2 = head2
    rows = B * N
    BLK = 2048
    hf = h.reshape(rows, C)
    out = pl.pallas_call(
        _head_body,
        grid=(rows // BLK,),
        in_specs=[
            pl.BlockSpec((BLK, C), lambda i: (i, 0)),
            pl.BlockSpec((C, W1.shape[1]), lambda i: (0, 0)),
            pl.BlockSpec((1, W1.shape[1]), lambda i: (0, 0)),
            pl.BlockSpec((1, W1.shape[1]), lambda i: (0, 0)),
            pl.BlockSpec((1, W1.shape[1]), lambda i: (0, 0)),
            pl.BlockSpec((C, W2.shape[1]), lambda i: (0, 0)),
            pl.BlockSpec((1, W2.shape[1]), lambda i: (0, 0)),
        ],
        out_specs=pl.BlockSpec((BLK, W2.shape[1]), lambda i: (i, 0)),
        out_shape=jax.ShapeDtypeStruct((rows, W2.shape[1]), jnp.float32),
        interpret=_INTERPRET,
    )(hf, W1, b1.reshape(1, -1), g1.reshape(1, -1), be1.reshape(1, -1),
      W2, b2.reshape(1, -1))
    return out.reshape(B, N, -1)


@jax.jit
def _forward(xyz, cls_label, params):
    B, C, N = xyz.shape
    x0 = jnp.transpose(xyz, (0, 2, 1))

    x1, f1 = _sa_level(x0, x0, 1024, [0.1, 0.2, 0.4], [32, 64, 128],
                       params['sa1'])
    x2, f2 = _sa_level(x1, f1, 512, [0.4, 0.8], [64, 128], params['sa2'])

    h = jnp.concatenate([x2, f2], -1)
    f3 = jnp.max(_mlp(params['sa3'], h), axis=1, keepdims=True)
    x3 = jnp.zeros((B, 1, 3), jnp.float32)

    f2 = _fp(x2, x3, f2, f3, params['fp3'])
    f1 = _fp(x1, x2, f1, f2, params['fp2'])

    cls_oh = jnp.broadcast_to(cls_label.reshape(B, 1, 1), (B, N, 1))
    p1 = jnp.concatenate([cls_oh, x0, x0], -1)
    f0 = _fp(x0, x1, p1, f1, params['fp1'])

    out = _head(f0, params['head1'], params['head2'])
    l3_points = jnp.transpose(f3, (0, 2, 1))
    return out, l3_points


def kernel(xyz, cls_label, params):
    return _forward(xyz, cls_label, params)
